# baseline (speedup 1.0000x reference)
import jax
import jax.numpy as jnp
from jax import lax
from jax.experimental import pallas as pl

B = 1024
D = 128
C = 128
H = 8
W = 8
S = H * W
P = 64
BB = 64
NB = B // BB
K3 = 3 * P


def _pass1_body(fm_ref, pos_ref, lab_ref, emb_ref, meansT_ref, me_ref,
                same_ref, dist_ref, val_ref, psum_ref):
    e = emb_ref[...]
    d2s = []
    for k in range(3):
        diff = e - me_ref[k:k + 1, :]
        d2s.append(jnp.sum(diff * diff, axis=1, keepdims=True))
    dist = jnp.sqrt(jnp.concatenate(d2s, axis=1))
    dist_ref[...] = dist
    d0 = dist[:, 0:1]
    d1 = dist[:, 1:2]
    d2 = dist[:, 2:3]
    val = jnp.where((d0 <= d1) & (d0 <= d2), 0,
                    jnp.where(d1 <= d2, 1, 2)).astype(jnp.int32)
    val_ref[...] = val

    meansT = meansT_ref[...]
    s_iota = lax.broadcasted_iota(jnp.int32, (S, P), 0)

    def body(i, carry):
        acc_dis, acc_same = carry
        fm_b = fm_ref[i]
        pos_b = pos_ref[pl.ds(i, 1), :]
        oh = (s_iota == pos_b).astype(jnp.float32)
        pe = jnp.dot(fm_b, oh, preferred_element_type=jnp.float32)
        pe3 = jnp.concatenate([pe, pe, pe], axis=1)
        diff = pe3 - meansT
        dis = jnp.sqrt(jnp.sum(diff * diff, axis=0, keepdims=True))
        lab = lab_ref[pl.ds(i, 1), :]
        s0 = dis[:, 0:P]
        s1 = dis[:, P:2 * P]
        s2 = dis[:, 2 * P:3 * P]
        same = jnp.where(lab == 0, s0, jnp.where(lab == 1, s1, s2))
        same_ref[pl.ds(i, 1), :] = same
        return (acc_dis + dis, acc_same + same)

    acc_dis, acc_same = lax.fori_loop(
        0, BB, body,
        (jnp.zeros((1, K3), jnp.float32), jnp.zeros((1, P), jnp.float32)))
    psum_ref[...] = jnp.concatenate([acc_dis, acc_same], axis=1)[None, :, :]


def _pass2_body(same_ref, psum_ref, loss_ref):
    ps = psum_ref[...]
    sum_dis = jnp.sum(ps[:, :, 0:K3])
    sum_same = jnp.sum(ps[:, :, K3:K3 + P])
    diff_mean = (sum_dis - sum_same) * (1.0 / (B * 2 * P))
    s = same_ref[...]
    t = jnp.maximum(s + (1.0 - diff_mean), 0.0)
    loss = jnp.sum(t) * (1.0 / (B * P))
    loss_ref[...] = loss.reshape(1, 1)


def kernel(labels, embeddings, feature_maps, means_b, means_m, means_n,
           centroids_x, centroids_y, mean_embedding_b, mean_embedding_m,
           mean_embedding_n):
    fm3 = feature_maps.reshape(B, C, S)
    pos = ((centroids_x // 28) * W + (centroids_y // 28)).T.astype(jnp.int32)
    labm = jnp.broadcast_to(labels.astype(jnp.int32)[:, None], (B, P))
    meansT = jnp.concatenate([means_b.T, means_m.T, means_n.T], axis=1)
    me = jnp.stack([mean_embedding_b, mean_embedding_m, mean_embedding_n])

    same, dist, val, psum = pl.pallas_call(
        _pass1_body,
        grid=(NB,),
        in_specs=[
            pl.BlockSpec((BB, C, S), lambda i: (i, 0, 0)),
            pl.BlockSpec((BB, P), lambda i: (i, 0)),
            pl.BlockSpec((BB, P), lambda i: (i, 0)),
            pl.BlockSpec((BB, D), lambda i: (i, 0)),
            pl.BlockSpec((C, K3), lambda i: (0, 0)),
            pl.BlockSpec((3, D), lambda i: (0, 0)),
        ],
        out_specs=[
            pl.BlockSpec((BB, P), lambda i: (i, 0)),
            pl.BlockSpec((BB, 3), lambda i: (i, 0)),
            pl.BlockSpec((BB, 1), lambda i: (i, 0)),
            pl.BlockSpec((1, 1, K3 + P), lambda i: (i, 0, 0)),
        ],
        out_shape=[
            jax.ShapeDtypeStruct((B, P), jnp.float32),
            jax.ShapeDtypeStruct((B, 3), jnp.float32),
            jax.ShapeDtypeStruct((B, 1), jnp.int32),
            jax.ShapeDtypeStruct((NB, 1, K3 + P), jnp.float32),
        ],
    )(fm3, pos, labm, embeddings, meansT, me)

    loss = pl.pallas_call(
        _pass2_body,
        out_shape=jax.ShapeDtypeStruct((1, 1), jnp.float32),
    )(same, psum)

    distances_im = dist.reshape(B, 3, 1)
    return (distances_im, val.reshape(B), loss[0, 0])


# speedup vs baseline: 3.3773x; 3.3773x over previous
import jax
import jax.numpy as jnp
from jax import lax
from jax.experimental import pallas as pl

B = 1024
D = 128
C = 128
H = 8
W = 8
S = H * W
P = 64
BB = 64
NB = B // BB
K3 = 3 * P


def _pass1_body(fm_ref, pos_ref, lab_ref, emb_ref, meansT_ref, me_ref,
                same_ref, dist_ref, val_ref, psum_ref):
    e = emb_ref[...]
    d2s = []
    for k in range(3):
        diff = e - me_ref[k:k + 1, :]
        d2s.append(jnp.sum(diff * diff, axis=1, keepdims=True))
    dist = jnp.sqrt(jnp.concatenate(d2s, axis=1))
    dist_ref[...] = dist
    d0 = dist[:, 0:1]
    d1 = dist[:, 1:2]
    d2 = dist[:, 2:3]
    val = jnp.where((d0 <= d1) & (d0 <= d2), 0,
                    jnp.where(d1 <= d2, 1, 2)).astype(jnp.int32)
    val_ref[...] = val

    fm = fm_ref[...]
    pos = pos_ref[...]
    idx3 = jnp.broadcast_to(pos[:, None, :], (BB, C, P))
    pe = jnp.take_along_axis(fm, idx3, axis=2)
    pe3 = jnp.concatenate([pe, pe, pe], axis=2)
    diff = pe3 - meansT_ref[...][None, :, :]
    dis = jnp.sqrt(jnp.sum(diff * diff, axis=1))
    lab = lab_ref[...]
    s0 = dis[:, 0:P]
    s1 = dis[:, P:2 * P]
    s2 = dis[:, 2 * P:3 * P]
    same = jnp.where(lab == 0, s0, jnp.where(lab == 1, s1, s2))
    same_ref[...] = same
    pd = jnp.sum(dis, axis=0, keepdims=True)
    psm = jnp.sum(same, axis=0, keepdims=True)
    psum_ref[...] = jnp.concatenate([pd, psm], axis=1)[None, :, :]


def _pass2_body(same_ref, psum_ref, loss_ref):
    ps = psum_ref[...]
    sum_dis = jnp.sum(ps[:, :, 0:K3])
    sum_same = jnp.sum(ps[:, :, K3:K3 + P])
    diff_mean = (sum_dis - sum_same) * (1.0 / (B * 2 * P))
    s = same_ref[...]
    t = jnp.maximum(s + (1.0 - diff_mean), 0.0)
    loss = jnp.sum(t) * (1.0 / (B * P))
    loss_ref[...] = loss.reshape(1, 1)


def kernel(labels, embeddings, feature_maps, means_b, means_m, means_n,
           centroids_x, centroids_y, mean_embedding_b, mean_embedding_m,
           mean_embedding_n):
    fm3 = feature_maps.reshape(B, C, S)
    pos = ((centroids_x // 28) * W + (centroids_y // 28)).T.astype(jnp.int32)
    labm = jnp.broadcast_to(labels.astype(jnp.int32)[:, None], (B, P))
    meansT = jnp.concatenate([means_b.T, means_m.T, means_n.T], axis=1)
    me = jnp.stack([mean_embedding_b, mean_embedding_m, mean_embedding_n])

    same, dist, val, psum = pl.pallas_call(
        _pass1_body,
        grid=(NB,),
        in_specs=[
            pl.BlockSpec((BB, C, S), lambda i: (i, 0, 0)),
            pl.BlockSpec((BB, P), lambda i: (i, 0)),
            pl.BlockSpec((BB, P), lambda i: (i, 0)),
            pl.BlockSpec((BB, D), lambda i: (i, 0)),
            pl.BlockSpec((C, K3), lambda i: (0, 0)),
            pl.BlockSpec((3, D), lambda i: (0, 0)),
        ],
        out_specs=[
            pl.BlockSpec((BB, P), lambda i: (i, 0)),
            pl.BlockSpec((BB, 3), lambda i: (i, 0)),
            pl.BlockSpec((BB, 1), lambda i: (i, 0)),
            pl.BlockSpec((1, 1, K3 + P), lambda i: (i, 0, 0)),
        ],
        out_shape=[
            jax.ShapeDtypeStruct((B, P), jnp.float32),
            jax.ShapeDtypeStruct((B, 3), jnp.float32),
            jax.ShapeDtypeStruct((B, 1), jnp.int32),
            jax.ShapeDtypeStruct((NB, 1, K3 + P), jnp.float32),
        ],
    )(fm3, pos, labm, embeddings, meansT, me)

    loss = pl.pallas_call(
        _pass2_body,
        out_shape=jax.ShapeDtypeStruct((1, 1), jnp.float32),
    )(same, psum)

    distances_im = dist.reshape(B, 3, 1)
    return (distances_im, val.reshape(B), loss[0, 0])


# speedup vs baseline: 3.4316x; 1.0161x over previous
import jax
import jax.numpy as jnp
from jax import lax
from jax.experimental import pallas as pl
from jax.experimental.pallas import tpu as pltpu

B = 1024
D = 128
C = 128
H = 8
W = 8
S = H * W
P = 64
BB = 64
NB = B // BB
CP = C // 2


def _body(fm_ref, comb_ref, emb_ref, means_ref, me_ref,
          dist_ref, val_ref, loss_ref,
          same_scr, accd_scr, accs_scr):
    pid = pl.program_id(0)

    @pl.when(pid < NB)
    def _main():
        e = emb_ref[...]
        d2s = []
        for k in range(3):
            de = e - me_ref[k:k + 1, :]
            d2s.append(jnp.sum(de * de, axis=1, keepdims=True))
        dist = jnp.sqrt(jnp.concatenate(d2s, axis=1))
        dist_ref[...] = dist
        da = dist[:, 0:1]
        db = dist[:, 1:2]
        dc = dist[:, 2:3]
        val = jnp.where((da <= db) & (da <= dc), 0,
                        jnp.where(db <= dc, 1, 2)).astype(jnp.int32)
        val_ref[...] = val

        comb = comb_ref[...]
        pos = comb & 63
        lab = comb >> 6
        idx2 = jnp.concatenate([pos, pos + S], axis=1)
        idx3 = jnp.broadcast_to(idx2[:, None, :], (BB, CP, 128))
        fm = fm_ref[...]
        pe = jnp.take_along_axis(fm, idx3, axis=2)

        dks = []
        for k in range(3):
            dm = pe - means_ref[k][None, :, :]
            t = jnp.sum(dm * dm, axis=1)
            dks.append(jnp.sqrt(t[:, 0:S] + t[:, S:2 * S]))
        same = jnp.where(lab == 0, dks[0],
                         jnp.where(lab == 1, dks[1], dks[2]))
        same_scr[pl.ds(pid * BB, BB), :] = same

        pd = jnp.sum(dks[0] + dks[1] + dks[2], axis=0, keepdims=True)
        ps = jnp.sum(same, axis=0, keepdims=True)

        @pl.when(pid == 0)
        def _init():
            accd_scr[...] = pd
            accs_scr[...] = ps

        @pl.when(pid > 0)
        def _acc():
            accd_scr[...] += pd
            accs_scr[...] += ps

    @pl.when(pid == NB)
    def _loss():
        sum_dis = jnp.sum(accd_scr[...])
        sum_same = jnp.sum(accs_scr[...])
        diff_mean = (sum_dis - sum_same) * (1.0 / (B * 2 * P))
        s = same_scr[...]
        t = jnp.maximum(s + (1.0 - diff_mean), 0.0)
        loss_ref[...] = (jnp.sum(t) * (1.0 / (B * P))).reshape(1, 1)


def kernel(labels, embeddings, feature_maps, means_b, means_m, means_n,
           centroids_x, centroids_y, mean_embedding_b, mean_embedding_m,
           mean_embedding_n):
    fm = feature_maps.reshape(B, CP, 128)
    comb = (((centroids_x // 28) * W + (centroids_y // 28)).T
            | (labels[:, None] << 6)).astype(jnp.int32)

    def _pair(m):
        return m.T.reshape(CP, 2, P).reshape(CP, 128)

    means3 = jnp.stack([_pair(means_b), _pair(means_m), _pair(means_n)])
    me = jnp.stack([mean_embedding_b, mean_embedding_m, mean_embedding_n])

    clamp = NB - 1
    dist, val, loss = pl.pallas_call(
        _body,
        grid=(NB + 1,),
        in_specs=[
            pl.BlockSpec((BB, CP, 128), lambda i: (jnp.minimum(i, clamp), 0, 0)),
            pl.BlockSpec((BB, P), lambda i: (jnp.minimum(i, clamp), 0)),
            pl.BlockSpec((BB, D), lambda i: (jnp.minimum(i, clamp), 0)),
            pl.BlockSpec((3, CP, 128), lambda i: (0, 0, 0)),
            pl.BlockSpec((3, D), lambda i: (0, 0)),
        ],
        out_specs=[
            pl.BlockSpec((BB, 3), lambda i: (jnp.minimum(i, clamp), 0)),
            pl.BlockSpec((BB, 1), lambda i: (jnp.minimum(i, clamp), 0)),
            pl.BlockSpec((1, 1), lambda i: (0, 0)),
        ],
        out_shape=[
            jax.ShapeDtypeStruct((B, 3), jnp.float32),
            jax.ShapeDtypeStruct((B, 1), jnp.int32),
            jax.ShapeDtypeStruct((1, 1), jnp.float32),
        ],
        scratch_shapes=[
            pltpu.VMEM((B, P), jnp.float32),
            pltpu.VMEM((1, P), jnp.float32),
            pltpu.VMEM((1, P), jnp.float32),
        ],
    )(fm, comb, embeddings, means3, me)

    return (dist.reshape(B, 3, 1), val.reshape(B), loss[0, 0])


# speedup vs baseline: 6.1755x; 1.7996x over previous
import jax
import jax.numpy as jnp
from jax import lax
from jax.experimental import pallas as pl
from jax.experimental.pallas import tpu as pltpu

B = 1024
D = 128
C = 128
H = 8
W = 8
S = H * W
P = 64
BB = 64
NB = B // BB
K3 = 3 * P


def _body(fm_ref, comb_ref, emb_ref, meansT_ref, me_ref,
          dist_ref, val_ref, loss_ref,
          same_scr, accd_scr, accs_scr):
    pid = pl.program_id(0)

    @pl.when(pid < NB)
    def _main():
        e = emb_ref[...]
        d2s = []
        for k in range(3):
            de = e - me_ref[k:k + 1, :]
            d2s.append(jnp.sum(de * de, axis=1, keepdims=True))
        dist = jnp.sqrt(jnp.concatenate(d2s, axis=1))
        dist_ref[...] = dist
        da = dist[:, 0:1]
        db = dist[:, 1:2]
        dc = dist[:, 2:3]
        val = jnp.where((da <= db) & (da <= dc), 0,
                        jnp.where(db <= dc, 1, 2)).astype(jnp.int32)
        val_ref[...] = val

        comb = comb_ref[...]
        pos = comb & 63
        lab = comb >> 6
        oh = (lax.broadcasted_iota(jnp.int32, (BB, S, P), 1)
              == pos[:, None, :]).astype(jnp.float32)
        fm = fm_ref[...]
        peT = lax.dot_general(fm, oh, (((1,), (1,)), ((0,), (0,))),
                              preferred_element_type=jnp.float32)
        pe3 = jnp.concatenate([peT, peT, peT], axis=2)
        diff = pe3 - meansT_ref[...][None, :, :]
        t = jnp.sum(diff * diff, axis=1)
        dis = jnp.sqrt(t)
        dks = [dis[:, 0:P], dis[:, P:2 * P], dis[:, 2 * P:3 * P]]
        same = jnp.where(lab == 0, dks[0],
                         jnp.where(lab == 1, dks[1], dks[2]))
        same_scr[pl.ds(pid * BB, BB), :] = same

        pd = jnp.sum(dks[0] + dks[1] + dks[2], axis=0, keepdims=True)
        ps = jnp.sum(same, axis=0, keepdims=True)

        @pl.when(pid == 0)
        def _init():
            accd_scr[...] = pd
            accs_scr[...] = ps

        @pl.when(pid > 0)
        def _acc():
            accd_scr[...] += pd
            accs_scr[...] += ps

    @pl.when(pid == NB)
    def _loss():
        sum_dis = jnp.sum(accd_scr[...])
        sum_same = jnp.sum(accs_scr[...])
        diff_mean = (sum_dis - sum_same) * (1.0 / (B * 2 * P))
        s = same_scr[...]
        t = jnp.maximum(s + (1.0 - diff_mean), 0.0)
        loss_ref[...] = (jnp.sum(t) * (1.0 / (B * P))).reshape(1, 1)


def kernel(labels, embeddings, feature_maps, means_b, means_m, means_n,
           centroids_x, centroids_y, mean_embedding_b, mean_embedding_m,
           mean_embedding_n):
    fm = jnp.transpose(feature_maps, (0, 2, 3, 1)).reshape(B, S, C)
    comb = (((centroids_x // 28) * W + (centroids_y // 28)).T
            | (labels[:, None] << 6)).astype(jnp.int32)
    meansT = jnp.concatenate([means_b.T, means_m.T, means_n.T], axis=1)
    me = jnp.stack([mean_embedding_b, mean_embedding_m, mean_embedding_n])

    clamp = NB - 1
    dist, val, loss = pl.pallas_call(
        _body,
        grid=(NB + 1,),
        in_specs=[
            pl.BlockSpec((BB, S, C), lambda i: (jnp.minimum(i, clamp), 0, 0)),
            pl.BlockSpec((BB, P), lambda i: (jnp.minimum(i, clamp), 0)),
            pl.BlockSpec((BB, D), lambda i: (jnp.minimum(i, clamp), 0)),
            pl.BlockSpec((C, K3), lambda i: (0, 0)),
            pl.BlockSpec((3, D), lambda i: (0, 0)),
        ],
        out_specs=[
            pl.BlockSpec((BB, 3), lambda i: (jnp.minimum(i, clamp), 0)),
            pl.BlockSpec((BB, 1), lambda i: (jnp.minimum(i, clamp), 0)),
            pl.BlockSpec((1, 1), lambda i: (0, 0)),
        ],
        out_shape=[
            jax.ShapeDtypeStruct((B, 3), jnp.float32),
            jax.ShapeDtypeStruct((B, 1), jnp.int32),
            jax.ShapeDtypeStruct((1, 1), jnp.float32),
        ],
        scratch_shapes=[
            pltpu.VMEM((B, P), jnp.float32),
            pltpu.VMEM((1, P), jnp.float32),
            pltpu.VMEM((1, P), jnp.float32),
        ],
    )(fm, comb, embeddings, meansT, me)

    return (dist.reshape(B, 3, 1), val.reshape(B), loss[0, 0])


# speedup vs baseline: 6.2035x; 1.0045x over previous
import jax
import jax.numpy as jnp
from jax import lax
from jax.experimental import pallas as pl
from jax.experimental.pallas import tpu as pltpu

B = 1024
D = 128
C = 128
H = 8
W = 8
S = H * W
P = 64
BB = 128
NB = B // BB
K3 = 3 * P


def _body(fm_ref, cx_ref, cy_ref, lab_ref, emb_ref, meansT_ref, me_ref,
          dist_ref, val_ref, loss_ref,
          same_scr, accd_scr, accs_scr):
    pid = pl.program_id(0)

    @pl.when(pid < NB)
    def _main():
        e = emb_ref[...]
        d2s = []
        for k in range(3):
            de = e - me_ref[k:k + 1, :]
            d2s.append(jnp.sum(de * de, axis=1, keepdims=True))
        dist = jnp.sqrt(jnp.concatenate(d2s, axis=1))
        dist_ref[...] = dist
        da = dist[:, 0:1]
        db = dist[:, 1:2]
        dc = dist[:, 2:3]
        val = jnp.where((da <= db) & (da <= dc), 0,
                        jnp.where(db <= dc, 1, 2)).astype(jnp.int32)
        val_ref[...] = val

        cx = cx_ref[...]
        cy = cy_ref[...]
        pos_t = ((cx * 2341) >> 16) * W + ((cy * 2341) >> 16)
        pos = jnp.transpose(pos_t)
        lab = jnp.broadcast_to(
            jnp.transpose(lab_ref[0]), (BB, P))

        oh = (lax.broadcasted_iota(jnp.int32, (BB, S, P), 1)
              == pos[:, None, :]).astype(jnp.float32)
        fm = fm_ref[...]
        peT = lax.dot_general(fm, oh, (((1,), (1,)), ((0,), (0,))),
                              preferred_element_type=jnp.float32)
        meansT = meansT_ref[...]
        dks = []
        for k in range(3):
            dk = peT - meansT[None, :, k * P:(k + 1) * P]
            dks.append(jnp.sqrt(jnp.sum(dk * dk, axis=1)))
        same = jnp.where(lab == 0, dks[0],
                         jnp.where(lab == 1, dks[1], dks[2]))
        same_scr[pl.ds(pid * BB, BB), :] = same

        pd = jnp.sum(dks[0] + dks[1] + dks[2], axis=0, keepdims=True)
        ps = jnp.sum(same, axis=0, keepdims=True)

        @pl.when(pid == 0)
        def _init():
            accd_scr[...] = pd
            accs_scr[...] = ps

        @pl.when(pid > 0)
        def _acc():
            accd_scr[...] += pd
            accs_scr[...] += ps

    @pl.when(pid == NB)
    def _loss():
        sum_dis = jnp.sum(accd_scr[...])
        sum_same = jnp.sum(accs_scr[...])
        diff_mean = (sum_dis - sum_same) * (1.0 / (B * 2 * P))
        s = same_scr[...]
        t = jnp.maximum(s + (1.0 - diff_mean), 0.0)
        loss_ref[...] = (jnp.sum(t) * (1.0 / (B * P))).reshape(1, 1)


def kernel(labels, embeddings, feature_maps, means_b, means_m, means_n,
           centroids_x, centroids_y, mean_embedding_b, mean_embedding_m,
           mean_embedding_n):
    fm = jnp.transpose(feature_maps, (0, 2, 3, 1)).reshape(B, S, C)
    lab3 = labels.astype(jnp.int32).reshape(NB, 1, BB)
    meansT = jnp.concatenate([means_b.T, means_m.T, means_n.T], axis=1)
    me = jnp.stack([mean_embedding_b, mean_embedding_m, mean_embedding_n])

    clamp = NB - 1
    dist, val, loss = pl.pallas_call(
        _body,
        grid=(NB + 1,),
        in_specs=[
            pl.BlockSpec((BB, S, C), lambda i: (jnp.minimum(i, clamp), 0, 0)),
            pl.BlockSpec((P, BB), lambda i: (0, jnp.minimum(i, clamp))),
            pl.BlockSpec((P, BB), lambda i: (0, jnp.minimum(i, clamp))),
            pl.BlockSpec((1, 1, BB), lambda i: (jnp.minimum(i, clamp), 0, 0)),
            pl.BlockSpec((BB, D), lambda i: (jnp.minimum(i, clamp), 0)),
            pl.BlockSpec((C, K3), lambda i: (0, 0)),
            pl.BlockSpec((3, D), lambda i: (0, 0)),
        ],
        out_specs=[
            pl.BlockSpec((BB, 3), lambda i: (jnp.minimum(i, clamp), 0)),
            pl.BlockSpec((BB, 1), lambda i: (jnp.minimum(i, clamp), 0)),
            pl.BlockSpec((1, 1), lambda i: (0, 0)),
        ],
        out_shape=[
            jax.ShapeDtypeStruct((B, 3), jnp.float32),
            jax.ShapeDtypeStruct((B, 1), jnp.int32),
            jax.ShapeDtypeStruct((1, 1), jnp.float32),
        ],
        scratch_shapes=[
            pltpu.VMEM((B, P), jnp.float32),
            pltpu.VMEM((1, P), jnp.float32),
            pltpu.VMEM((1, P), jnp.float32),
        ],
    )(fm, centroids_x, centroids_y, lab3, embeddings, meansT, me)

    return (dist.reshape(B, 3, 1), val.reshape(B), loss[0, 0])
